# speedup vs baseline: 1.0563x; 1.0563x over previous
import jax
import jax.numpy as jnp
from jax.experimental import pallas as pl

NUM_POS_FEATS = 256


def _body(col_ref, row_ref, out_ref):
    h = out_ref.shape[1]
    w = out_ref.shape[2]
    half = NUM_POS_FEATS // 2
    col = col_ref[:w, :]
    row = row_ref[:h, :]
    out_ref[0, :, :, :half] = jnp.broadcast_to(col[None, :, :], (h, w, half))
    out_ref[0, :, :, half:] = jnp.broadcast_to(row[:, None, :], (h, w, half))


def kernel(tensor_list, row_embed, col_embed):
    b, h, w = tensor_list.shape[0], tensor_list.shape[-3], tensor_list.shape[-2]
    out = pl.pallas_call(
        _body,
        grid=(b,),
        in_specs=[
            pl.BlockSpec(col_embed.shape, lambda i: (0, 0)),
            pl.BlockSpec(row_embed.shape, lambda i: (0, 0)),
        ],
        out_specs=pl.BlockSpec((1, h, w, NUM_POS_FEATS), lambda i: (i, 0, 0, 0)),
        out_shape=jax.ShapeDtypeStruct((b, h, w, NUM_POS_FEATS), jnp.float32),
    )(col_embed, row_embed)
    return out


# speedup vs baseline: 1.1312x; 1.0709x over previous
import jax
import jax.numpy as jnp
from jax.experimental import pallas as pl
from jax.experimental.pallas import tpu as pltpu

NUM_POS_FEATS = 256


def _make_body(b, h, w):
    half = NUM_POS_FEATS // 2

    def _body(col_ref, row_ref, out_ref, scratch, sem):
        col = col_ref[:w, :]
        row = row_ref[:h, :]
        scratch[:, :, :half] = jnp.broadcast_to(col[None, :, :], (h, w, half))
        scratch[:, :, half:] = jnp.broadcast_to(row[:, None, :], (h, w, half))
        copies = [
            pltpu.make_async_copy(scratch, out_ref.at[i], sem) for i in range(b)
        ]
        for c in copies:
            c.start()
        for c in copies:
            c.wait()

    return _body


def kernel(tensor_list, row_embed, col_embed):
    b, h, w = tensor_list.shape[0], tensor_list.shape[-3], tensor_list.shape[-2]
    out = pl.pallas_call(
        _make_body(b, h, w),
        in_specs=[
            pl.BlockSpec(memory_space=pltpu.VMEM),
            pl.BlockSpec(memory_space=pltpu.VMEM),
        ],
        out_specs=pl.BlockSpec(memory_space=pl.ANY),
        out_shape=jax.ShapeDtypeStruct((b, h, w, NUM_POS_FEATS), jnp.float32),
        scratch_shapes=[
            pltpu.VMEM((h, w, NUM_POS_FEATS), jnp.float32),
            pltpu.SemaphoreType.DMA,
        ],
    )(col_embed, row_embed)
    return out
